# baseline (device time: 115022 ns/iter reference)
import jax
import jax.numpy as jnp
from jax import lax
from jax.experimental import pallas as pl
from jax.experimental.pallas import tpu as pltpu


def kernel(x):
    m, n = x.shape
    h = m // 2
    C = 32
    rc = h // C

    def body(
        x_ref,
        out_ref,
        xf32_ref,
        xh_ref,
        xrecv_ref,
        sum_ref,
        load_sems,
        store_sems,
        xsend_sems,
        xrecv_sems,
        ysend_sems,
        yrecv_sems,
    ):
        my_x = lax.axis_index("x")
        my_y = lax.axis_index("y")
        half = my_y * h
        x_nbr = (1 - my_x, my_y)
        y_nbr = (my_x, 1 - my_y)

        def load_copy(k):
            return pltpu.make_async_copy(
                x_ref.at[pl.ds(half + k * rc, rc), :],
                xf32_ref.at[pl.ds(k * rc, rc), :],
                load_sems.at[k],
            )

        def store_copy(k):
            return pltpu.make_async_copy(
                sum_ref.at[pl.ds(k * rc, rc), :],
                out_ref.at[pl.ds(half + k * rc, rc), :],
                store_sems.at[k],
            )

        def x_rdma(k):
            return pltpu.make_async_remote_copy(
                src_ref=xh_ref.at[pl.ds(k * rc, rc), :],
                dst_ref=xrecv_ref.at[pl.ds(k * rc, rc), :],
                send_sem=xsend_sems.at[k],
                recv_sem=xrecv_sems.at[k],
                device_id=x_nbr,
                device_id_type=pl.DeviceIdType.MESH,
            )

        def y_rdma(k):
            return pltpu.make_async_remote_copy(
                src_ref=sum_ref.at[pl.ds(k * rc, rc), :],
                dst_ref=out_ref.at[pl.ds(half + k * rc, rc), :],
                send_sem=ysend_sems.at[k],
                recv_sem=yrecv_sems.at[k],
                device_id=y_nbr,
                device_id_type=pl.DeviceIdType.MESH,
            )

        for k in range(C):
            load_copy(k).start()

        barrier_sem = pltpu.get_barrier_semaphore()
        for nbr in (x_nbr, y_nbr):
            pl.semaphore_signal(
                barrier_sem, inc=1,
                device_id=nbr, device_id_type=pl.DeviceIdType.MESH,
            )
        pl.semaphore_wait(barrier_sem, 2)

        for k in range(C):
            load_copy(k).wait()
            xh_ref[pl.ds(k * rc, rc), :] = xf32_ref[
                pl.ds(k * rc, rc), :
            ].astype(jnp.bfloat16)
            x_rdma(k).start()

        for k in range(C):
            r = x_rdma(k)
            r.wait_recv()
            sum_ref[pl.ds(k * rc, rc), :] = (
                xh_ref[pl.ds(k * rc, rc), :] + xrecv_ref[pl.ds(k * rc, rc), :]
            )
            y_rdma(k).start()
            store_copy(k).start()

        for k in range(C):
            x_rdma(k).wait_send()
            r = y_rdma(k)
            r.wait_send()
            r.wait_recv()
            store_copy(k).wait()

    return pl.pallas_call(
        body,
        out_shape=jax.ShapeDtypeStruct((m, n), jnp.bfloat16),
        in_specs=[pl.BlockSpec(memory_space=pl.ANY)],
        out_specs=pl.BlockSpec(memory_space=pl.ANY),
        scratch_shapes=[
            pltpu.VMEM((h, n), jnp.float32),
            pltpu.VMEM((h, n), jnp.bfloat16),
            pltpu.VMEM((h, n), jnp.bfloat16),
            pltpu.VMEM((h, n), jnp.bfloat16),
            pltpu.SemaphoreType.DMA((C,)),
            pltpu.SemaphoreType.DMA((C,)),
            pltpu.SemaphoreType.DMA((C,)),
            pltpu.SemaphoreType.DMA((C,)),
            pltpu.SemaphoreType.DMA((C,)),
            pltpu.SemaphoreType.DMA((C,)),
        ],
        compiler_params=pltpu.CompilerParams(
            collective_id=0, vmem_limit_bytes=64 * 1024 * 1024
        ),
    )(x)


# device time: 114911 ns/iter; 1.0010x vs baseline; 1.0010x over previous
import jax
import jax.numpy as jnp
from jax import lax
from jax.experimental import pallas as pl
from jax.experimental.pallas import tpu as pltpu


def kernel(x):
    m, n = x.shape
    h = m // 2
    sizes = [64] + [128] * 31 + [64]
    offs = [sum(sizes[:i]) for i in range(len(sizes))]
    chunks = list(zip(offs, sizes))
    C = len(chunks)

    def body(
        x_ref,
        out_ref,
        xf32_ref,
        xh_ref,
        xrecv_ref,
        sum_ref,
        load_sems,
        store_sems,
        xsend_sems,
        xrecv_sems,
        ysend_sems,
        yrecv_sems,
    ):
        my_x = lax.axis_index("x")
        my_y = lax.axis_index("y")
        half = my_y * h
        x_nbr = (1 - my_x, my_y)
        y_nbr = (my_x, 1 - my_y)

        def load_copy(k):
            off, sz = chunks[k]
            return pltpu.make_async_copy(
                x_ref.at[pl.ds(half + off, sz), :],
                xf32_ref.at[pl.ds(off, sz), :],
                load_sems.at[k],
            )

        def store_copy(k):
            off, sz = chunks[k]
            return pltpu.make_async_copy(
                sum_ref.at[pl.ds(off, sz), :],
                out_ref.at[pl.ds(half + off, sz), :],
                store_sems.at[k],
            )

        def x_rdma(k):
            off, sz = chunks[k]
            return pltpu.make_async_remote_copy(
                src_ref=xh_ref.at[pl.ds(off, sz), :],
                dst_ref=xrecv_ref.at[pl.ds(off, sz), :],
                send_sem=xsend_sems.at[k],
                recv_sem=xrecv_sems.at[k],
                device_id=x_nbr,
                device_id_type=pl.DeviceIdType.MESH,
            )

        def y_rdma(k):
            off, sz = chunks[k]
            return pltpu.make_async_remote_copy(
                src_ref=sum_ref.at[pl.ds(off, sz), :],
                dst_ref=out_ref.at[pl.ds(half + off, sz), :],
                send_sem=ysend_sems.at[k],
                recv_sem=yrecv_sems.at[k],
                device_id=y_nbr,
                device_id_type=pl.DeviceIdType.MESH,
            )

        for k in range(C):
            load_copy(k).start()

        barrier_sem = pltpu.get_barrier_semaphore()
        for nbr in (x_nbr, y_nbr):
            pl.semaphore_signal(
                barrier_sem, inc=1,
                device_id=nbr, device_id_type=pl.DeviceIdType.MESH,
            )
        pl.semaphore_wait(barrier_sem, 2)

        for k in range(C):
            off, sz = chunks[k]
            load_copy(k).wait()
            xh_ref[pl.ds(off, sz), :] = xf32_ref[
                pl.ds(off, sz), :
            ].astype(jnp.bfloat16)
            x_rdma(k).start()

        for k in range(C):
            off, sz = chunks[k]
            r = x_rdma(k)
            r.wait_recv()
            sum_ref[pl.ds(off, sz), :] = (
                xh_ref[pl.ds(off, sz), :] + xrecv_ref[pl.ds(off, sz), :]
            )
            y_rdma(k).start()
            store_copy(k).start()

        for k in range(C):
            x_rdma(k).wait_send()
            r = y_rdma(k)
            r.wait_send()
            r.wait_recv()
            store_copy(k).wait()

    return pl.pallas_call(
        body,
        out_shape=jax.ShapeDtypeStruct((m, n), jnp.bfloat16),
        in_specs=[pl.BlockSpec(memory_space=pl.ANY)],
        out_specs=pl.BlockSpec(memory_space=pl.ANY),
        scratch_shapes=[
            pltpu.VMEM((h, n), jnp.float32),
            pltpu.VMEM((h, n), jnp.bfloat16),
            pltpu.VMEM((h, n), jnp.bfloat16),
            pltpu.VMEM((h, n), jnp.bfloat16),
            pltpu.SemaphoreType.DMA((C,)),
            pltpu.SemaphoreType.DMA((C,)),
            pltpu.SemaphoreType.DMA((C,)),
            pltpu.SemaphoreType.DMA((C,)),
            pltpu.SemaphoreType.DMA((C,)),
            pltpu.SemaphoreType.DMA((C,)),
        ],
        compiler_params=pltpu.CompilerParams(
            collective_id=0, vmem_limit_bytes=64 * 1024 * 1024
        ),
    )(x)
